# Initial kernel scaffold; baseline (speedup 1.0000x reference)
import jax
import jax.numpy as jnp
from jax.experimental import pallas as pl


def kernel(features, edge_index, edge_weight, kernel, bias, skip_weight):
    raise NotImplementedError("write your pallas kernel here")


# speedup vs baseline: 4.0614x; 4.0614x over previous
import functools

import jax
import jax.numpy as jnp
from jax import lax
from jax.experimental import pallas as pl
from jax.experimental.pallas import tpu as pltpu
from jax.experimental.pallas import tpu_sc as plsc

N_NODES = 10000
N_EDGES = 320000
D_FEAT = 128
N_CHANNELS = 128

NC = 2
NS = 16
L = 16

CH = N_CHANNELS // NC
EDGE_CHUNK = 80
CHUNK_ROWS = N_EDGES // EDGE_CHUNK
TILE_CHUNKS = CHUNK_ROWS // NS
ROWS_PER_TILE = N_NODES // NS
SLAB = 125
N_SLABS = ROWS_PER_TILE // SLAB

SELU_SCALE = 1.0507009873554805
SELU_ALPHA = 1.6732632423543772


def _matmul_body(f_ref, w_ref, b_ref, o_ref):
    o_ref[...] = (
        jnp.dot(f_ref[...], w_ref[...], preferred_element_type=jnp.float32)
        + b_ref[...]
    )


def _transform(features, weight, bias):
    blk = 1000
    grid = (N_NODES // blk,)
    return pl.pallas_call(
        _matmul_body,
        grid=grid,
        in_specs=[
            pl.BlockSpec((blk, D_FEAT), lambda i: (i, 0)),
            pl.BlockSpec((D_FEAT, N_CHANNELS), lambda i: (0, 0)),
            pl.BlockSpec((1, N_CHANNELS), lambda i: (0, 0)),
        ],
        out_specs=pl.BlockSpec((blk, N_CHANNELS), lambda i: (i, 0)),
        out_shape=jax.ShapeDtypeStruct((N_NODES, N_CHANNELS), jnp.float32),
    )(features, weight, bias.reshape(1, N_CHANNELS))


def _sc_body(table_hbm, src_hbm, dst_hbm, ew_hbm, skip_hbm, out_hbm,
             src_v, dst_v, ew_v, rows_v, buf_v, skip_v, acc_sh, sem):
    c = lax.axis_index("c")
    s = lax.axis_index("s")
    zeros = jnp.zeros((L,), jnp.float32)

    def _zrow(r, _):
        for t in range(CH // L):
            buf_v[r, pl.ds(t * L, L)] = zeros
        return _
    lax.fori_loop(0, SLAB, _zrow, None)
    for q in range(N_SLABS):
        pltpu.sync_copy(buf_v, acc_sh.at[pl.ds(s * ROWS_PER_TILE + q * SLAB, SLAB)])

    pltpu.sync_copy(src_hbm.at[pl.ds(s * TILE_CHUNKS, TILE_CHUNKS)], src_v)
    pltpu.sync_copy(dst_hbm.at[pl.ds(s * TILE_CHUNKS, TILE_CHUNKS)], dst_v)
    pltpu.sync_copy(ew_hbm.at[pl.ds(s * TILE_CHUNKS, TILE_CHUNKS)], ew_v)

    def _fix(r, _):
        for t in range(EDGE_CHUNK // L):
            sl = pl.ds(t * L, L)
            src_v[r, sl] = src_v[r, sl] * 2 + c
        return _
    lax.fori_loop(0, TILE_CHUNKS, _fix, None)

    plsc.subcore_barrier()

    def _chunk(j, _):
        pltpu.async_copy(table_hbm.at[src_v.at[j]], rows_v, sem).wait()
        jfull = jnp.full((L,), j, jnp.int32)

        def _scale(k, __):
            w = plsc.load_gather(ew_v, [jfull, jnp.full((L,), k, jnp.int32)])
            for t in range(CH // L):
                sl = pl.ds(t * L, L)
                rows_v[k, sl] = rows_v[k, sl] * w
            return __
        lax.fori_loop(0, EDGE_CHUNK, _scale, None)

        pltpu.sync_copy(rows_v, acc_sh.at[dst_v.at[j]], add=True)
        return _
    lax.fori_loop(0, TILE_CHUNKS, _chunk, None)

    plsc.subcore_barrier()

    pltpu.sync_copy(skip_hbm.at[pl.ds(c * CH, CH)], skip_v)
    for q in range(N_SLABS):
        row0 = s * ROWS_PER_TILE + q * SLAB
        pltpu.sync_copy(acc_sh.at[pl.ds(row0, SLAB)], buf_v)

        def _selu_row(r, _):
            for t in range(CH // L):
                sl = pl.ds(t * L, L)
                x = buf_v[r, sl] + skip_v[sl]
                y = SELU_SCALE * jnp.where(
                    x > 0.0, x, SELU_ALPHA * (jnp.exp(x) - 1.0))
                buf_v[r, sl] = y
            return _
        lax.fori_loop(0, SLAB, _selu_row, None)

        pltpu.sync_copy(buf_v, out_hbm.at[pl.ds(row0, SLAB), pl.ds(c * CH, CH)])


@functools.partial(
    pl.kernel,
    mesh=plsc.VectorSubcoreMesh(core_axis_name="c", subcore_axis_name="s"),
    out_type=jax.ShapeDtypeStruct((N_NODES, N_CHANNELS), jnp.float32),
    compiler_params=pltpu.CompilerParams(
        use_tc_tiling_on_sc=False, needs_layout_passes=False),
    scratch_types=[
        pltpu.VMEM((TILE_CHUNKS, EDGE_CHUNK), jnp.int32),
        pltpu.VMEM((TILE_CHUNKS, EDGE_CHUNK), jnp.int32),
        pltpu.VMEM((TILE_CHUNKS, EDGE_CHUNK), jnp.float32),
        pltpu.VMEM((EDGE_CHUNK, CH), jnp.float32),
        pltpu.VMEM((SLAB, CH), jnp.float32),
        pltpu.VMEM((CH,), jnp.float32),
        pltpu.VMEM_SHARED((N_NODES, CH), jnp.float32),
        pltpu.SemaphoreType.DMA,
    ],
)
def _sc_aggregate(table_hbm, src_hbm, dst_hbm, ew_hbm, skip_hbm, out_hbm,
                  src_v, dst_v, ew_v, rows_v, buf_v, skip_v, acc_sh, sem):
    _sc_body(table_hbm, src_hbm, dst_hbm, ew_hbm, skip_hbm, out_hbm,
             src_v, dst_v, ew_v, rows_v, buf_v, skip_v, acc_sh, sem)


def kernel(features, edge_index, edge_weight, kernel, bias, skip_weight):
    out = _transform(features, kernel, bias)
    table = out.reshape(2 * N_NODES, CH)
    src = edge_index[0].astype(jnp.int32).reshape(CHUNK_ROWS, EDGE_CHUNK)
    dst = edge_index[1].astype(jnp.int32).reshape(CHUNK_ROWS, EDGE_CHUNK)
    ew = edge_weight.reshape(CHUNK_ROWS, EDGE_CHUNK)
    return _sc_aggregate(table, src, dst, ew, skip_weight)


# speedup vs baseline: 6.6547x; 1.6385x over previous
import functools

import jax
import jax.numpy as jnp
from jax import lax
from jax.experimental import pallas as pl
from jax.experimental.pallas import tpu as pltpu
from jax.experimental.pallas import tpu_sc as plsc

N_NODES = 10000
N_EDGES = 320000
D_FEAT = 128
N_CHANNELS = 128

NC = 2
NS = 16
L = 16

CH = N_CHANNELS // NC
EDGE_CHUNK = 80
CHUNK_ROWS = N_EDGES // EDGE_CHUNK
TILE_CHUNKS = CHUNK_ROWS // NS
NBUF = 5
N_STAGES = 2
STAGE = TILE_CHUNKS // N_STAGES
ROWS_PER_TILE = N_NODES // NS
SLAB = 125
N_SLABS = ROWS_PER_TILE // SLAB

SELU_SCALE = 1.0507009873554805
SELU_ALPHA = 1.6732632423543772


def _matmul_body(f_ref, w_ref, b_ref, o_ref):
    o_ref[...] = (
        jnp.dot(f_ref[...], w_ref[...], preferred_element_type=jnp.float32)
        + b_ref[...]
    )


def _transform(features, weight, bias):
    blk = 1000
    grid = (N_NODES // blk,)
    return pl.pallas_call(
        _matmul_body,
        grid=grid,
        in_specs=[
            pl.BlockSpec((blk, D_FEAT), lambda i: (i, 0)),
            pl.BlockSpec((D_FEAT, N_CHANNELS), lambda i: (0, 0)),
            pl.BlockSpec((1, N_CHANNELS), lambda i: (0, 0)),
        ],
        out_specs=pl.BlockSpec((blk, N_CHANNELS), lambda i: (i, 0)),
        out_shape=jax.ShapeDtypeStruct((N_NODES, N_CHANNELS), jnp.float32),
    )(features, weight, bias.reshape(1, N_CHANNELS))


def _sc_body(table_hbm, src_hbm, dst_hbm, ew_hbm, skip_hbm, out_hbm,
             src_v, dst_v, ew_v, rows, buf_v, skip_v, acc_sh, sems):
    c = lax.axis_index("c")
    s = lax.axis_index("s")
    zeros = jnp.zeros((L,), jnp.float32)

    def _zrow(r, _):
        for t in range(CH // L):
            buf_v[r, pl.ds(t * L, L)] = zeros
        return _
    lax.fori_loop(0, SLAB, _zrow, None)
    for q in range(N_SLABS):
        pltpu.sync_copy(buf_v, acc_sh.at[pl.ds(s * ROWS_PER_TILE + q * SLAB, SLAB)])

    plsc.subcore_barrier()

    def _scale(j, r):
        jfull = jnp.full((L,), j, jnp.int32)

        def _edge(k, __):
            w = plsc.load_gather(ew_v, [jfull, jnp.full((L,), k, jnp.int32)])
            for t in range(CH // L):
                sl = pl.ds(t * L, L)
                rows[r][k, sl] = rows[r][k, sl] * w
            return __
        lax.fori_loop(0, EDGE_CHUNK, _edge, None)

    n_blk = STAGE // NBUF
    for h in range(N_STAGES):
        base = s * TILE_CHUNKS + h * STAGE
        pltpu.sync_copy(src_hbm.at[pl.ds(base, STAGE)], src_v)
        pltpu.sync_copy(dst_hbm.at[pl.ds(base, STAGE)], dst_v)
        pltpu.sync_copy(ew_hbm.at[pl.ds(base, STAGE)], ew_v)

        def _fix(r, _):
            for t in range(EDGE_CHUNK // L):
                sl = pl.ds(t * L, L)
                src_v[r, sl] = src_v[r, sl] * 2 + c
            return _
        lax.fori_loop(0, STAGE, _fix, None)

        for j in range(NBUF - 1):
            pltpu.async_copy(table_hbm.at[src_v.at[j]], rows[j], sems[j])

        def _block(i, _):
            for b in range(NBUF):
                j = i * NBUF + b
                pltpu.make_async_copy(
                    table_hbm.at[src_v.at[j]], rows[b], sems[b]).wait()
                _scale(j, b)
                pltpu.sync_copy(rows[b], acc_sh.at[dst_v.at[j]], add=True)
                jn = j + NBUF - 1
                bn = (b + NBUF - 1) % NBUF
                if b == 0:
                    pltpu.async_copy(
                        table_hbm.at[src_v.at[jn]], rows[bn], sems[bn])
                else:
                    @pl.when(i < n_blk - 1)
                    def _():
                        pltpu.async_copy(
                            table_hbm.at[src_v.at[jn]], rows[bn], sems[bn])
            return _
        lax.fori_loop(0, n_blk, _block, None)

    plsc.subcore_barrier()

    pltpu.sync_copy(skip_hbm.at[pl.ds(c * CH, CH)], skip_v)
    for q in range(N_SLABS):
        row0 = s * ROWS_PER_TILE + q * SLAB
        pltpu.sync_copy(acc_sh.at[pl.ds(row0, SLAB)], buf_v)

        def _selu_row(r, _):
            for t in range(CH // L):
                sl = pl.ds(t * L, L)
                x = buf_v[r, sl] + skip_v[sl]
                y = SELU_SCALE * jnp.where(
                    x > 0.0, x, SELU_ALPHA * (jnp.exp(x) - 1.0))
                buf_v[r, sl] = y
            return _
        lax.fori_loop(0, SLAB, _selu_row, None)

        pltpu.sync_copy(buf_v, out_hbm.at[pl.ds(row0, SLAB), pl.ds(c * CH, CH)])


@functools.partial(
    pl.kernel,
    mesh=plsc.VectorSubcoreMesh(core_axis_name="c", subcore_axis_name="s"),
    out_type=jax.ShapeDtypeStruct((N_NODES, N_CHANNELS), jnp.float32),
    compiler_params=pltpu.CompilerParams(
        use_tc_tiling_on_sc=False, needs_layout_passes=False),
    scratch_types=[
        pltpu.VMEM((STAGE, EDGE_CHUNK), jnp.int32),
        pltpu.VMEM((STAGE, EDGE_CHUNK), jnp.int32),
        pltpu.VMEM((STAGE, EDGE_CHUNK), jnp.float32),
        *[pltpu.VMEM((EDGE_CHUNK, CH), jnp.float32) for _ in range(NBUF)],
        pltpu.VMEM((SLAB, CH), jnp.float32),
        pltpu.VMEM((CH,), jnp.float32),
        pltpu.VMEM_SHARED((N_NODES, CH), jnp.float32),
        *[pltpu.SemaphoreType.DMA for _ in range(NBUF)],
    ],
)
def _sc_aggregate(table_hbm, src_hbm, dst_hbm, ew_hbm, skip_hbm, out_hbm,
                  src_v, dst_v, ew_v, r0, r1, r2, r3, r4, buf_v, skip_v,
                  acc_sh, s0, s1, s2, s3, s4):
    _sc_body(table_hbm, src_hbm, dst_hbm, ew_hbm, skip_hbm, out_hbm,
             src_v, dst_v, ew_v, [r0, r1, r2, r3, r4], buf_v, skip_v, acc_sh,
             [s0, s1, s2, s3, s4])


def kernel(features, edge_index, edge_weight, kernel, bias, skip_weight):
    out = _transform(features, kernel, bias)
    table = out.reshape(2 * N_NODES, CH)
    src = edge_index[0].astype(jnp.int32).reshape(CHUNK_ROWS, EDGE_CHUNK)
    dst = edge_index[1].astype(jnp.int32).reshape(CHUNK_ROWS, EDGE_CHUNK)
    ew = edge_weight.reshape(CHUNK_ROWS, EDGE_CHUNK)
    return _sc_aggregate(table, src, dst, ew, skip_weight)


# speedup vs baseline: 8.5522x; 1.2851x over previous
import functools

import jax
import jax.numpy as jnp
from jax import lax
from jax.experimental import pallas as pl
from jax.experimental.pallas import tpu as pltpu
from jax.experimental.pallas import tpu_sc as plsc

N_NODES = 10000
N_EDGES = 320000
D_FEAT = 128
N_CHANNELS = 128

NC = 2
NS = 16
L = 16

CH = N_CHANNELS // NC
EDGE_CHUNK = 80
CHUNK_ROWS = N_EDGES // EDGE_CHUNK
TILE_CHUNKS = CHUNK_ROWS // NS
NBUF = 5
N_STAGES = 2
STAGE = TILE_CHUNKS // N_STAGES
ROWS_PER_TILE = N_NODES // NS
SLAB = 125
N_SLABS = ROWS_PER_TILE // SLAB

SELU_SCALE = 1.0507009873554805
SELU_ALPHA = 1.6732632423543772


def _matmul_body(f_ref, w_ref, b_ref, o_ref):
    o_ref[...] = (
        jnp.dot(f_ref[...], w_ref[...], preferred_element_type=jnp.float32)
        + b_ref[...]
    )


def _transform(features, weight, bias):
    blk = 1000
    grid = (N_NODES // blk,)
    return pl.pallas_call(
        _matmul_body,
        grid=grid,
        in_specs=[
            pl.BlockSpec((blk, D_FEAT), lambda i: (i, 0)),
            pl.BlockSpec((D_FEAT, N_CHANNELS), lambda i: (0, 0)),
            pl.BlockSpec((1, N_CHANNELS), lambda i: (0, 0)),
        ],
        out_specs=pl.BlockSpec((blk, N_CHANNELS), lambda i: (i, 0)),
        out_shape=jax.ShapeDtypeStruct((N_NODES, N_CHANNELS), jnp.float32),
    )(features, weight, bias.reshape(1, N_CHANNELS))


def _sc_body(table_hbm, src_hbm, dst_hbm, ew_hbm, skip_hbm, out_hbm,
             src_v, dst_v, ew_v, rows, buf_v, skip_v, acc_sh, sems, ssems):
    c = lax.axis_index("c")
    s = lax.axis_index("s")
    zeros = jnp.zeros((L,), jnp.float32)

    def _zrow(r, _):
        for t in range(CH // L):
            buf_v[r, pl.ds(t * L, L)] = zeros
        return _
    lax.fori_loop(0, SLAB, _zrow, None)
    for q in range(N_SLABS):
        pltpu.sync_copy(buf_v, acc_sh.at[pl.ds(s * ROWS_PER_TILE + q * SLAB, SLAB)])

    plsc.subcore_barrier()

    def _scale(j, r):
        jfull = jnp.full((L,), j, jnp.int32)

        def _edge(k, __):
            w = plsc.load_gather(ew_v, [jfull, jnp.full((L,), k, jnp.int32)])
            for t in range(CH // L):
                sl = pl.ds(t * L, L)
                rows[r][k, sl] = rows[r][k, sl] * w
            return __
        lax.fori_loop(0, EDGE_CHUNK, _edge, None, unroll=4)

    DEPTH = NBUF - 2
    n_blk = STAGE // NBUF
    for h in range(N_STAGES):
        base = s * TILE_CHUNKS + h * STAGE
        pltpu.sync_copy(src_hbm.at[pl.ds(base, STAGE)], src_v)
        pltpu.sync_copy(dst_hbm.at[pl.ds(base, STAGE)], dst_v)
        pltpu.sync_copy(ew_hbm.at[pl.ds(base, STAGE)], ew_v)

        def _fix(r, _):
            for t in range(EDGE_CHUNK // L):
                sl = pl.ds(t * L, L)
                src_v[r, sl] = src_v[r, sl] * 2 + c
            return _
        lax.fori_loop(0, STAGE, _fix, None)

        for j in range(DEPTH):
            pltpu.async_copy(table_hbm.at[src_v.at[j]], rows[j], sems[j])

        def _block(i, _):
            for b in range(NBUF):
                j = i * NBUF + b
                pltpu.make_async_copy(
                    table_hbm.at[src_v.at[j]], rows[b], sems[b]).wait()
                _scale(j, b)
                pltpu.async_copy(
                    rows[b], acc_sh.at[dst_v.at[j]], ssems[b], add=True)
                jn = j + DEPTH
                bn = (b + DEPTH) % NBUF

                def _prefetch():
                    pltpu.make_async_copy(
                        rows[bn], acc_sh.at[dst_v.at[j]], ssems[bn]).wait()
                    pltpu.async_copy(
                        table_hbm.at[src_v.at[jn]], rows[bn], sems[bn])

                if b < NBUF - DEPTH:
                    @pl.when(i > 0)
                    def _():
                        _prefetch()

                    @pl.when(i == 0)
                    def _():
                        pltpu.async_copy(
                            table_hbm.at[src_v.at[jn]], rows[bn], sems[bn])
                else:
                    @pl.when(i < n_blk - 1)
                    def _():
                        _prefetch()
            return _
        lax.fori_loop(0, n_blk, _block, None)

        for b in range(NBUF):
            pltpu.make_async_copy(
                rows[b], acc_sh.at[dst_v.at[0]], ssems[b]).wait()

    plsc.subcore_barrier()

    pltpu.sync_copy(skip_hbm.at[pl.ds(c * CH, CH)], skip_v)
    for q in range(N_SLABS):
        row0 = s * ROWS_PER_TILE + q * SLAB
        pltpu.sync_copy(acc_sh.at[pl.ds(row0, SLAB)], buf_v)

        def _selu_row(r, _):
            for t in range(CH // L):
                sl = pl.ds(t * L, L)
                x = buf_v[r, sl] + skip_v[sl]
                y = SELU_SCALE * jnp.where(
                    x > 0.0, x, SELU_ALPHA * (jnp.exp(x) - 1.0))
                buf_v[r, sl] = y
            return _
        lax.fori_loop(0, SLAB, _selu_row, None)

        pltpu.sync_copy(buf_v, out_hbm.at[pl.ds(row0, SLAB), pl.ds(c * CH, CH)])


@functools.partial(
    pl.kernel,
    mesh=plsc.VectorSubcoreMesh(core_axis_name="c", subcore_axis_name="s"),
    out_type=jax.ShapeDtypeStruct((N_NODES, N_CHANNELS), jnp.float32),
    compiler_params=pltpu.CompilerParams(
        use_tc_tiling_on_sc=False, needs_layout_passes=False),
    scratch_types=[
        pltpu.VMEM((STAGE, EDGE_CHUNK), jnp.int32),
        pltpu.VMEM((STAGE, EDGE_CHUNK), jnp.int32),
        pltpu.VMEM((STAGE, EDGE_CHUNK), jnp.float32),
        *[pltpu.VMEM((EDGE_CHUNK, CH), jnp.float32) for _ in range(NBUF)],
        pltpu.VMEM((SLAB, CH), jnp.float32),
        pltpu.VMEM((CH,), jnp.float32),
        pltpu.VMEM_SHARED((N_NODES, CH), jnp.float32),
        *[pltpu.SemaphoreType.DMA for _ in range(2 * NBUF)],
    ],
)
def _sc_aggregate(table_hbm, src_hbm, dst_hbm, ew_hbm, skip_hbm, out_hbm,
                  src_v, dst_v, ew_v, r0, r1, r2, r3, r4, buf_v, skip_v,
                  acc_sh, s0, s1, s2, s3, s4, t0, t1, t2, t3, t4):
    _sc_body(table_hbm, src_hbm, dst_hbm, ew_hbm, skip_hbm, out_hbm,
             src_v, dst_v, ew_v, [r0, r1, r2, r3, r4], buf_v, skip_v, acc_sh,
             [s0, s1, s2, s3, s4], [t0, t1, t2, t3, t4])


def kernel(features, edge_index, edge_weight, kernel, bias, skip_weight):
    out = _transform(features, kernel, bias)
    table = out.reshape(2 * N_NODES, CH)
    src = edge_index[0].astype(jnp.int32).reshape(CHUNK_ROWS, EDGE_CHUNK)
    dst = edge_index[1].astype(jnp.int32).reshape(CHUNK_ROWS, EDGE_CHUNK)
    ew = edge_weight.reshape(CHUNK_ROWS, EDGE_CHUNK)
    return _sc_aggregate(table, src, dst, ew, skip_weight)
